# baseline (speedup 1.0000x reference)
import functools

import jax
import jax.numpy as jnp
from jax import lax
from jax.experimental import pallas as pl
from jax.experimental.pallas import tpu as pltpu
from jax.experimental.pallas import tpu_sc as plsc

NUM_CLASSES = 1000000
EMB_CHANNEL = 64
BATCH = 16384

NC = 2
NS = 16
NW = NC * NS
B_PER_W = BATCH // NW


@functools.lru_cache(maxsize=1)
def _build_gather():
    mesh = plsc.VectorSubcoreMesh(core_axis_name="c", subcore_axis_name="s")

    @functools.partial(
        pl.kernel,
        mesh=mesh,
        out_type=jax.ShapeDtypeStruct((BATCH, EMB_CHANNEL), jnp.float32),
        scratch_types=[
            pltpu.VMEM_SHARED((NS, B_PER_W), jnp.int32),
            pltpu.SMEM((B_PER_W,), jnp.int32),
            pltpu.VMEM((B_PER_W, EMB_CHANNEL), jnp.float32),
            pltpu.SemaphoreType.DMA,
        ],
    )
    def emb_gather(table_hbm, idx_hbm, out_hbm, idx_sh, idx_s, rows_v, sem):
        cid = lax.axis_index("c")
        sid = lax.axis_index("s")
        wid = sid * NC + cid
        base = wid * B_PER_W
        pltpu.sync_copy(idx_hbm.at[pl.ds(base, B_PER_W)], idx_sh.at[sid])
        pltpu.sync_copy(idx_sh.at[sid], idx_s)

        def body(j8, _):
            j = j8 * 8
            for u in range(8):
                pltpu.async_copy(
                    table_hbm.at[idx_s[j + u]], rows_v.at[j + u], sem
                )
            return _

        lax.fori_loop(0, B_PER_W // 8, body, None)
        pltpu.make_async_copy(
            out_hbm.at[pl.ds(base, B_PER_W)], rows_v, sem
        ).wait()
        pltpu.sync_copy(rows_v, out_hbm.at[pl.ds(base, B_PER_W)])

    return emb_gather


def kernel(class_labels, label_emb_weight):
    idx = class_labels.astype(jnp.int32)
    return _build_gather()(label_emb_weight, idx)
